# baseline (speedup 1.0000x reference)
import jax
import jax.numpy as jnp
from jax import lax
from jax.experimental import pallas as pl
from jax.experimental.pallas import tpu as pltpu
from jax.experimental.pallas import tpu_sc as plsc

RADIUS = 2.0
R2 = RADIUS * RADIUS
KNN = 32
DFEAT = 32
DOUT = DFEAT + 3
NLANE = 16
NCORES = 2
NSUB = 16
NWORKERS = NCORES * NSUB
GCH = 128


def _make_sc_kernel(n, nq, qpt, nbatch):
    nstep = n // NLANE
    nchunk = qpt * KNN // GCH
    qrow = GCH // KNN
    rows_pt = qpt * KNN

    def body(candf, xyzf, feat2, out, qf, xyzraw, idxq, idx2d, fbuf, packed,
             sem):
        wid = lax.axis_index("s") * NCORES + lax.axis_index("c")
        b = wid // (NWORKERS // nbatch)
        pltpu.sync_copy(candf.at[pl.ds(wid * qpt * 3, qpt * 3)],
                        qf.at[pl.ds(0, qpt * 3)])
        pltpu.sync_copy(xyzf.at[pl.ds(b * n * 3, n * 3)], xyzraw)
        rowbase = b * n
        lanes = lax.iota(jnp.int32, NLANE)
        lanes3 = lanes * 3
        dnums = lax.GatherDimensionNumbers(
            offset_dims=(), collapsed_slice_dims=(0,), start_index_map=(0,))

        def _splat(vec, j):
            sel = jnp.full((NLANE, 1), j, jnp.int32)
            return lax.gather(vec, sel, dnums, (1,),
                              mode=lax.GatherScatterMode.PROMISE_IN_BOUNDS)

        def per_query(qi, carry):
            w = qf[pl.ds(qi * 3, NLANE)]
            qx = _splat(w, 0)
            qy = _splat(w, 1)
            qz = _splat(w, 2)

            def cond(c):
                i, cnt = c
                return jnp.logical_and(cnt < KNN, i < nstep)

            def step(c):
                i, cnt = c
                base = lanes3 + i * (NLANE * 3)
                dx = plsc.load_gather(xyzraw, [base]) - qx
                dy = plsc.load_gather(xyzraw, [base + 1]) - qy
                dz = plsc.load_gather(xyzraw, [base + 2]) - qz
                d = dx * dx + dy * dy + dz * dz
                m = d <= R2
                plsc.store_compressed(
                    idxq.at[pl.ds(cnt, NLANE)],
                    lanes + (i * NLANE + rowbase), mask=m)
                return i + jnp.int32(1), cnt + jnp.sum(m.astype(jnp.int32))

            _, cnt = lax.while_loop(
                cond, step, (jnp.int32(0), jnp.int32(0)))
            v0 = idxq[pl.ds(0, NLANE)]
            v1 = idxq[pl.ds(NLANE, NLANE)]
            first = jnp.where(cnt > 0, _splat(v0, 0),
                              jnp.full((NLANE,), rowbase + (n - 1),
                                       jnp.int32))
            o0 = jnp.where(lanes < cnt, v0, first)
            o1 = jnp.where(lanes + NLANE < cnt, v1, first)
            r = qi // qrow
            c0 = (qi % qrow) * KNN
            idx2d[r, pl.ds(c0, NLANE)] = o0
            idx2d[r, pl.ds(c0 + NLANE, NLANE)] = o1
            return carry

        lax.fori_loop(0, qpt, per_query, 0)

        for j in range(nchunk):
            pltpu.async_copy(feat2.at[idx2d.at[j]], fbuf, sem).wait()

            def mrow(r, c):
                r2 = j * GCH + r
                packed[r2, pl.ds(3, NLANE)] = fbuf[r, pl.ds(0, NLANE)]
                packed[r2, pl.ds(3 + NLANE, NLANE)] = fbuf[r, pl.ds(NLANE,
                                                                    NLANE)]
                return c

            lax.fori_loop(0, GCH, mrow, 0)
            for t in range(GCH // NLANE):
                iv = idx2d[j, pl.ds(t * NLANE, NLANE)]
                loc3 = (iv - rowbase) * 3
                rvec = lanes + (j * GCH + t * NLANE)
                for cc in range(3):
                    v = plsc.load_gather(xyzraw, [loc3 + cc]) * (1.0 / RADIUS)
                    plsc.store_scatter(
                        packed, [rvec, jnp.full((NLANE,), cc, jnp.int32)], v)
        pltpu.sync_copy(packed, out.at[pl.ds(wid * rows_pt, rows_pt)])

    mesh = plsc.VectorSubcoreMesh(
        core_axis_name="c", subcore_axis_name="s",
        num_cores=NCORES, num_subcores=NSUB)
    return pl.kernel(
        body,
        out_type=jax.ShapeDtypeStruct((nq * KNN, DOUT), jnp.float32),
        mesh=mesh,
        compiler_params=pltpu.CompilerParams(
            needs_layout_passes=False, use_tc_tiling_on_sc=False),
        scratch_types=[
            pltpu.VMEM((qpt * 3 + NLANE,), jnp.float32),
            pltpu.VMEM((n * 3,), jnp.float32),
            pltpu.VMEM((KNN + NLANE,), jnp.int32),
            pltpu.VMEM((nchunk, GCH), jnp.int32),
            pltpu.VMEM((GCH, DFEAT), jnp.float32),
            pltpu.VMEM((rows_pt, DOUT), jnp.float32),
            pltpu.SemaphoreType.DMA,
        ],
    )


def kernel(candidate_pts, src_keypts, tgt_pts_xyz, tgt_deep_feat_pts):
    del src_keypts
    b, ktop, c, _ = candidate_pts.shape
    s = ktop * c
    n = tgt_pts_xyz.shape[1]
    nq = b * s
    qpt = nq // NWORKERS

    candf = candidate_pts.reshape(nq * 3)
    xyzf = tgt_pts_xyz.reshape(b * n * 3)
    feat2 = tgt_deep_feat_pts.reshape(b * n, DFEAT)
    out = _make_sc_kernel(n, nq, qpt, b)(candf, xyzf, feat2)
    return out.reshape(b, ktop, c, KNN, DOUT)


# speedup vs baseline: 1.0743x; 1.0743x over previous
import jax
import jax.numpy as jnp
from jax import lax
from jax.experimental import pallas as pl
from jax.experimental.pallas import tpu as pltpu
from jax.experimental.pallas import tpu_sc as plsc

RADIUS = 2.0
R2 = RADIUS * RADIUS
KNN = 32
DFEAT = 32
DOUT = DFEAT + 3
NLANE = 16
NCORES = 2
NSUB = 16
NWORKERS = NCORES * NSUB
GCH = 128


def _make_sc_kernel(n, nq, qpt, nbatch):
    nstep = n // NLANE
    nchunk = qpt * KNN // GCH
    qrow = GCH // KNN
    rows_pt = qpt * KNN

    def body(candf, xyzf, feat2, out, qf, xyzraw, idxq, idx2d, fbuf_a,
             fbuf_b, packed, sem):
        wid = lax.axis_index("s") * NCORES + lax.axis_index("c")
        b = wid // (NWORKERS // nbatch)
        pltpu.sync_copy(candf.at[pl.ds(wid * qpt * 3, qpt * 3)],
                        qf.at[pl.ds(0, qpt * 3)])
        pltpu.sync_copy(xyzf.at[pl.ds(b * n * 3, n * 3)], xyzraw)
        rowbase = b * n
        lanes = lax.iota(jnp.int32, NLANE)
        lanes3 = lanes * 3
        dnums = lax.GatherDimensionNumbers(
            offset_dims=(), collapsed_slice_dims=(0,), start_index_map=(0,))

        def _splat(vec, j):
            sel = jnp.full((NLANE, 1), j, jnp.int32)
            return lax.gather(vec, sel, dnums, (1,),
                              mode=lax.GatherScatterMode.PROMISE_IN_BOUNDS)

        def per_query(qi, carry):
            w = qf[pl.ds(qi * 3, NLANE)]
            qx = _splat(w, 0)
            qy = _splat(w, 1)
            qz = _splat(w, 2)

            def cond(c):
                i, cnt = c
                return jnp.logical_and(cnt < KNN, i < nstep)

            def step(c):
                i, cnt = c
                base = lanes3 + i * (NLANE * 3)
                dx = plsc.load_gather(xyzraw, [base]) - qx
                dy = plsc.load_gather(xyzraw, [base + 1]) - qy
                dz = plsc.load_gather(xyzraw, [base + 2]) - qz
                d = dx * dx + dy * dy + dz * dz
                m = d <= R2
                plsc.store_compressed(
                    idxq.at[pl.ds(cnt, NLANE)],
                    lanes + (i * NLANE + rowbase), mask=m)
                return i + jnp.int32(1), cnt + jnp.sum(m.astype(jnp.int32))

            _, cnt = lax.while_loop(
                cond, step, (jnp.int32(0), jnp.int32(0)))
            v0 = idxq[pl.ds(0, NLANE)]
            v1 = idxq[pl.ds(NLANE, NLANE)]
            first = jnp.where(cnt > 0, _splat(v0, 0),
                              jnp.full((NLANE,), rowbase + (n - 1),
                                       jnp.int32))
            o0 = jnp.where(lanes < cnt, v0, first)
            o1 = jnp.where(lanes + NLANE < cnt, v1, first)
            r = qi // qrow
            c0 = (qi % qrow) * KNN
            idx2d[r, pl.ds(c0, NLANE)] = o0
            idx2d[r, pl.ds(c0 + NLANE, NLANE)] = o1
            return carry

        lax.fori_loop(0, qpt, per_query, 0)

        fbufs = [fbuf_a, fbuf_b]
        cps = [None, None]
        cps[0] = pltpu.async_copy(feat2.at[idx2d.at[0]], fbufs[0], sem)
        for j in range(nchunk):
            cps[j % 2].wait()
            if j + 1 < nchunk:
                cps[(j + 1) % 2] = pltpu.async_copy(
                    feat2.at[idx2d.at[j + 1]], fbufs[(j + 1) % 2], sem)
            fb = fbufs[j % 2]

            def mrow(r, c):
                r2 = j * GCH + r
                q = r2 >> 5
                k = r2 & (KNN - 1)
                packed[q, k, pl.ds(3, NLANE)] = fb[r, pl.ds(0, NLANE)]
                packed[q, k, pl.ds(3 + NLANE, NLANE)] = fb[r, pl.ds(NLANE,
                                                                    NLANE)]
                return c

            lax.fori_loop(0, GCH, mrow, 0)
            for t in range(GCH // NLANE):
                iv = idx2d[j, pl.ds(t * NLANE, NLANE)]
                loc3 = (iv - rowbase) * 3
                rvec = lanes + (j * GCH + t * NLANE)
                qv = rvec >> 5
                kv = rvec & (KNN - 1)
                for cc in range(3):
                    v = plsc.load_gather(xyzraw, [loc3 + cc]) * (1.0 / RADIUS)
                    plsc.store_scatter(
                        packed, [qv, kv, jnp.full((NLANE,), cc, jnp.int32)],
                        v)
        pltpu.sync_copy(packed, out.at[pl.ds(wid * qpt, qpt)])

    mesh = plsc.VectorSubcoreMesh(
        core_axis_name="c", subcore_axis_name="s",
        num_cores=NCORES, num_subcores=NSUB)
    return pl.kernel(
        body,
        out_type=jax.ShapeDtypeStruct((nq, KNN, DOUT), jnp.float32),
        mesh=mesh,
        compiler_params=pltpu.CompilerParams(
            needs_layout_passes=False, use_tc_tiling_on_sc=False),
        scratch_types=[
            pltpu.VMEM((qpt * 3 + NLANE,), jnp.float32),
            pltpu.VMEM((n * 3,), jnp.float32),
            pltpu.VMEM((KNN + NLANE,), jnp.int32),
            pltpu.VMEM((nchunk, GCH), jnp.int32),
            pltpu.VMEM((GCH, DFEAT), jnp.float32),
            pltpu.VMEM((GCH, DFEAT), jnp.float32),
            pltpu.VMEM((qpt, KNN, DOUT), jnp.float32),
            pltpu.SemaphoreType.DMA,
        ],
    )


def kernel(candidate_pts, src_keypts, tgt_pts_xyz, tgt_deep_feat_pts):
    del src_keypts
    b, ktop, c, _ = candidate_pts.shape
    s = ktop * c
    n = tgt_pts_xyz.shape[1]
    nq = b * s
    qpt = nq // NWORKERS

    candf = candidate_pts.reshape(nq * 3)
    xyzf = tgt_pts_xyz.reshape(b * n * 3)
    feat2 = tgt_deep_feat_pts.reshape(b * n, DFEAT)
    out = _make_sc_kernel(n, nq, qpt, b)(candf, xyzf, feat2)
    return out.reshape(b, ktop, c, KNN, DOUT)
